# Initial kernel scaffold; baseline (speedup 1.0000x reference)
import functools

import jax
import jax.numpy as jnp
from jax import lax
from jax.experimental import pallas as pl
from jax.experimental.pallas import tpu as pltpu
from jax.experimental.pallas import tpu_sc as plsc

N_BINS = 8
N_CODES = 256

NC = 2
NS = 16
L = 16
NW = NC * NS

N_IDX = 128 * 8192
PER_W = N_IDX // NW
CHUNK = 4096
N_CHUNKS = PER_W // CHUNK


def _build_sc_lookup():
    mesh = plsc.VectorSubcoreMesh(
        core_axis_name="c", subcore_axis_name="s", num_cores=NC,
        num_subcores=NS,
    )

    @functools.partial(
        pl.kernel,
        out_type=jax.ShapeDtypeStruct((N_IDX * N_BINS,), jnp.float32),
        mesh=mesh,
        scratch_types=[
            pltpu.VMEM((N_CODES * N_BINS,), jnp.float32),
            pltpu.VMEM((CHUNK,), jnp.int32),
            pltpu.VMEM((CHUNK * N_BINS,), jnp.float32),
        ],
    )
    def sc_lookup(idx_hbm, cb_hbm, out_hbm, cb_v, idx_v, out_v):
        wid = lax.axis_index("s") * NC + lax.axis_index("c")
        base = wid * PER_W

        pltpu.sync_copy(cb_hbm, cb_v)

        lane = lax.iota(jnp.int32, L)
        pair = lane >> 3
        binpat = lane & 7

        def do_chunk(i, carry):
            cbase = base + i * CHUNK
            pltpu.sync_copy(idx_hbm.at[pl.ds(cbase, CHUNK)], idx_v)

            def do_vreg(j, carry2):
                gidx = pair + 2 * j
                idxv = plsc.load_gather(idx_v, [gidx])
                cbidx = (idxv << 3) | binpat
                rowv = plsc.load_gather(cb_v, [cbidx])
                out_v[pl.ds(L * j, L)] = rowv
                return carry2

            lax.fori_loop(0, CHUNK // 2, do_vreg, 0, unroll=4)
            pltpu.sync_copy(out_v, out_hbm.at[pl.ds(cbase * N_BINS,
                                                    CHUNK * N_BINS)])
            return carry

        lax.fori_loop(0, N_CHUNKS, do_chunk, 0)

    return sc_lookup


_sc_lookup = _build_sc_lookup()


def kernel(idx, codebook):
    out_flat = _sc_lookup(idx.reshape(-1), codebook.reshape(-1))
    return out_flat.reshape(idx.shape + (2, N_BINS // 2))


# speedup vs baseline: 1.1174x; 1.1174x over previous
import functools

import jax
import jax.numpy as jnp
from jax import lax
from jax.experimental import pallas as pl
from jax.experimental.pallas import tpu as pltpu
from jax.experimental.pallas import tpu_sc as plsc

N_BINS = 8
N_CODES = 256

NC = 2
NS = 16
L = 16
NW = NC * NS

N_IDX = 128 * 8192
PER_W = N_IDX // NW
CHUNK = 4096
N_CHUNKS = PER_W // CHUNK


def _build_sc_lookup():
    mesh = plsc.VectorSubcoreMesh(
        core_axis_name="c", subcore_axis_name="s", num_cores=NC,
        num_subcores=NS,
    )

    @functools.partial(
        pl.kernel,
        out_type=jax.ShapeDtypeStruct((N_IDX * N_BINS,), jnp.float32),
        mesh=mesh,
        scratch_types=[
            pltpu.VMEM((N_CODES * N_BINS,), jnp.float32),
            pltpu.VMEM((CHUNK,), jnp.int32),
            pltpu.VMEM((CHUNK * N_BINS,), jnp.float32),
        ],
        compiler_params=pltpu.CompilerParams(needs_layout_passes=False),
    )
    def sc_lookup(idx_hbm, cb_hbm, out_hbm, cb_v, idx_v, out_v):
        wid = lax.axis_index("s") * NC + lax.axis_index("c")
        base = wid * PER_W

        pltpu.sync_copy(cb_hbm, cb_v)

        lane = lax.iota(jnp.int32, L)
        pair = lane >> 3
        binpat = lane & 7

        def do_chunk(i, carry):
            cbase = base + i * CHUNK
            pltpu.sync_copy(idx_hbm.at[pl.ds(cbase, CHUNK)], idx_v)

            def do_vreg(j, carry2):
                gidx = pair + 2 * j
                idxv = plsc.load_gather(idx_v, [gidx])
                cbidx = (idxv << 3) | binpat
                rowv = plsc.load_gather(cb_v, [cbidx])
                out_v[pl.ds(L * j, L)] = rowv
                return carry2

            lax.fori_loop(0, CHUNK // 2, do_vreg, 0, unroll=4)
            pltpu.sync_copy(out_v, out_hbm.at[pl.ds(cbase * N_BINS,
                                                    CHUNK * N_BINS)])
            return carry

        lax.fori_loop(0, N_CHUNKS, do_chunk, 0)

    return sc_lookup


_sc_lookup = _build_sc_lookup()


def kernel(idx, codebook):
    out_flat = _sc_lookup(idx.reshape(-1), codebook.reshape(-1))
    return out_flat.reshape(idx.shape + (2, N_BINS // 2))
